# Initial kernel scaffold; baseline (speedup 1.0000x reference)
import functools

import jax
import jax.numpy as jnp
from jax.experimental import pallas as pl

C = 8
G = 8
BR = 4
M = 256
DEPTH = 7
N_TREES = 4
PER = (BR ** (DEPTH + 1) - 1) // (BR - 1)
CG = C * G
LEVEL_SIZES = [BR ** l for l in range(DEPTH + 1)]
LEVEL_STARTS = [(BR ** l - 1) // (BR - 1) for l in range(DEPTH + 1)]
LEAF_CHUNK = 4096


def _fll_kernel(la_ref, lb_ref, lpi_ref, *rest):
    x_refs = rest[:DEPTH + 1]
    out_ref = rest[DEPTH + 1]
    f32 = jnp.float32

    la = jax.nn.softmax(la_ref[...], axis=1)
    bt = jax.nn.softmax(lb_ref[...], axis=0)
    pi = jax.nn.softmax(lpi_ref[...], axis=1)
    p0 = pi.reshape(1, CG)

    r64 = jax.lax.broadcasted_iota(jnp.int32, (CG, CG), 0)
    c64 = jax.lax.broadcasted_iota(jnp.int32, (CG, CG), 1)
    bmask = (r64 // C == c64 // C).astype(f32)
    bds = []
    for p in range(BR):
        tp = la[p * CG:(p + 1) * CG, :]
        bds.append(jnp.tile(tp, (1, C)) * bmask)
    d_mat = jnp.concatenate(bds, axis=1)
    zero64 = jnp.zeros((CG, CG), f32)
    u_rows = []
    for p in range(BR):
        row = [zero64] * BR
        row[p] = bds[p].T
        u_rows.append(jnp.concatenate(row, axis=1))
    u_mat = jnp.concatenate(u_rows, axis=0)

    s_mat = (jax.lax.broadcasted_iota(jnp.int32, (CG, G), 0) // C
             == jax.lax.broadcasted_iota(jnp.int32, (CG, G), 1)).astype(f32)
    sb_mat = (jax.lax.broadcasted_iota(jnp.int32, (G, CG), 0)
              == jax.lax.broadcasted_iota(jnp.int32, (G, CG), 1) // C).astype(f32)

    dot = functools.partial(jnp.dot, precision=jax.lax.Precision.HIGHEST,
                            preferred_element_type=f32)

    priors = [p0]
    for l in range(1, DEPTH + 1):
        priors.append(dot(priors[-1], d_mat).reshape(LEVEL_SIZES[l], CG))

    def emit(x_col, n):
        oh = (x_col == jax.lax.broadcasted_iota(jnp.int32, (n, M), 1)).astype(f32)
        return dot(oh, bt)

    def prod4(u):
        return (u[:, 0:CG] * u[:, CG:2 * CG]
                * u[:, 2 * CG:3 * CG] * u[:, 3 * CG:4 * CG])

    for t in range(N_TREES):
        acc = jnp.zeros((1, G), f32)
        n7 = LEVEL_SIZES[DEPTH]
        pparts = []
        for cs in range(0, n7, LEAF_CHUNK):
            b = emit(x_refs[DEPTH][t, cs:cs + LEAF_CHUNK, :], LEAF_CHUNK)
            m = priors[DEPTH][cs:cs + LEAF_CHUNK, :] * b
            nu = dot(m, s_mat)
            acc = acc + jnp.sum(jnp.log(nu), axis=0, keepdims=True)
            q = b * dot(1.0 / nu, sb_mat)
            u = dot(q.reshape(LEAF_CHUNK // BR, BR * CG), u_mat)
            pparts.append(prod4(u))
        pprod = jnp.concatenate(pparts, axis=0)
        for l in range(DEPTH - 1, 0, -1):
            n = LEVEL_SIZES[l]
            b = emit(x_refs[l][t], n)
            m = priors[l] * b
            unnorm = m * m * pprod
            nu = dot(unnorm, s_mat)
            acc = acc + jnp.sum(jnp.log(nu), axis=0, keepdims=True)
            q = (m * b * pprod) * dot(1.0 / nu, sb_mat)
            u = dot(q.reshape(n // BR, BR * CG), u_mat)
            pprod = prod4(u)
        b = emit(x_refs[0][t], 1)
        m = p0 * b
        nu = dot(m * m * pprod, s_mat)
        acc = acc + jnp.log(nu)
        out_ref[t:t + 1, :] = acc


def kernel(lambda_A, lambda_B, lambda_Pi, x):
    la = jnp.transpose(lambda_A, (2, 3, 1, 0)).reshape(BR * CG, C)
    lb = jnp.transpose(lambda_B, (1, 2, 0)).reshape(M, CG)
    lpi = jnp.transpose(lambda_Pi, (1, 0))
    x2 = x.reshape(N_TREES, PER).astype(jnp.int32)
    xs = [x2[:, LEVEL_STARTS[l]:LEVEL_STARTS[l] + LEVEL_SIZES[l]]
          .reshape(N_TREES, LEVEL_SIZES[l], 1) for l in range(DEPTH + 1)]
    return pl.pallas_call(
        _fll_kernel,
        out_shape=jax.ShapeDtypeStruct((N_TREES, G), jnp.float32),
    )(la, lb, lpi, *xs)


# speedup vs baseline: 170.7338x; 170.7338x over previous
import functools

import jax
import jax.numpy as jnp
from jax.experimental import pallas as pl

C = 8
G = 8
BR = 4
M = 256
DEPTH = 7
N_TREES = 4
PER = (BR ** (DEPTH + 1) - 1) // (BR - 1)
CG = C * G
LEVEL_SIZES = [BR ** l for l in range(DEPTH + 1)]
LEVEL_STARTS = [(BR ** l - 1) // (BR - 1) for l in range(DEPTH + 1)]


def _fll_kernel(la_ref, lb_ref, lpi_ref, *rest):
    x_refs = rest[:DEPTH + 1]
    out_ref = rest[DEPTH + 1]
    f32 = jnp.float32

    la = jax.nn.softmax(la_ref[...], axis=1)
    bt = jax.nn.softmax(lb_ref[...], axis=0)
    pi = jax.nn.softmax(lpi_ref[...], axis=1)

    r64 = jax.lax.broadcasted_iota(jnp.int32, (CG, CG), 0)
    c64 = jax.lax.broadcasted_iota(jnp.int32, (CG, CG), 1)
    bmask = (r64 // C == c64 // C).astype(f32)
    bds = []
    for p in range(BR):
        tp = la[p * CG:(p + 1) * CG, :]
        bds.append(jnp.tile(tp, (1, C)) * bmask)
    bdst = [b.T for b in bds]

    s_mat = (jax.lax.broadcasted_iota(jnp.int32, (CG, G), 0) // C
             == jax.lax.broadcasted_iota(jnp.int32, (CG, G), 1)).astype(f32)
    sb_mat = (jax.lax.broadcasted_iota(jnp.int32, (G, CG), 0)
              == jax.lax.broadcasted_iota(jnp.int32, (G, CG), 1) // C).astype(f32)

    dot = functools.partial(jnp.dot, precision=jax.lax.Precision.HIGHEST,
                            preferred_element_type=f32)

    pmask = (jax.lax.broadcasted_iota(jnp.int32, (G, CG), 1) // C
             == jax.lax.broadcasted_iota(jnp.int32, (G, CG), 0)).astype(f32)
    p0 = dot(jnp.ones((1, G), f32), jnp.tile(pi, (1, C)) * pmask)

    priors = [p0]
    for l in range(1, DEPTH + 1):
        priors.append(jnp.concatenate(
            [dot(priors[-1], bds[p]) for p in range(BR)], axis=0))

    def emit(x_row, n):
        oh_t = (x_row == jax.lax.broadcasted_iota(jnp.int32, (M, n), 0)).astype(f32)
        return jax.lax.dot_general(oh_t, bt, (((0,), (0,)), ((), ())),
                                   precision=jax.lax.Precision.HIGHEST,
                                   preferred_element_type=f32)

    def tree_body(t, carry):
        acc = jnp.zeros((1, G), f32)
        pprod = None
        for l in range(DEPTH, 0, -1):
            npa = LEVEL_SIZES[l - 1]
            if l < DEPTH:
                b_full = emit(x_refs[l][t], LEVEL_SIZES[l])
            pnext = None
            for p in range(BR):
                if l == DEPTH:
                    b = emit(x_refs[l][t][:, p * npa:(p + 1) * npa], npa)
                else:
                    b = b_full[p * npa:(p + 1) * npa, :]
                m = priors[l][p * npa:(p + 1) * npa, :] * b
                if l == DEPTH:
                    unnorm = m
                    qb = b
                else:
                    pp = pprod[p * npa:(p + 1) * npa, :]
                    unnorm = m * m * pp
                    qb = m * b * pp
                nu = dot(unnorm, s_mat)
                acc = acc + jnp.sum(jnp.log(nu), axis=0, keepdims=True)
                q = qb * dot(1.0 / nu, sb_mat)
                u = dot(q, bdst[p])
                pnext = u if pnext is None else pnext * u
            pprod = pnext
        b = emit(x_refs[0][t], 1)
        m = p0 * b
        nu = dot(m * m * pprod, s_mat)
        acc = acc + jnp.log(nu)
        out_ref[pl.ds(t, 1), :] = acc
        return carry

    jax.lax.fori_loop(0, N_TREES, tree_body, 0)


def kernel(lambda_A, lambda_B, lambda_Pi, x):
    la = jnp.transpose(lambda_A, (2, 3, 1, 0)).reshape(BR * CG, C)
    lb = jnp.transpose(lambda_B, (1, 2, 0)).reshape(M, CG)
    lpi = jnp.transpose(lambda_Pi, (1, 0))
    x2 = x.reshape(N_TREES, PER).astype(jnp.int32)
    xs = []
    for l in range(DEPTH + 1):
        xl = x2[:, LEVEL_STARTS[l]:LEVEL_STARTS[l] + LEVEL_SIZES[l]]
        xl = xl.reshape((N_TREES,) + (BR,) * l)
        xl = jnp.transpose(xl, (0,) + tuple(range(l, 0, -1)))
        xs.append(xl.reshape(N_TREES, 1, LEVEL_SIZES[l]))
    return pl.pallas_call(
        _fll_kernel,
        out_shape=jax.ShapeDtypeStruct((N_TREES, G), jnp.float32),
    )(la, lb, lpi, *xs)
